# baseline (speedup 1.0000x reference)
import functools

import jax
import jax.numpy as jnp
from jax import lax
from jax.experimental import pallas as pl
from jax.experimental.pallas import tpu as pltpu
from jax.experimental.pallas import tpu_sc as plsc

N = 10000
E = 640000
H = 32
L = 32
G = 64

NC = 2
NS = 16
NW = NC * NS
CHUNK = 128
N_PAD = 10112
SLAB = N_PAD // NS
SPAN_MAX = 1024
SIZES = ([159 * 128, 159 * 128] + [156 * 128] * 13 + [154 * 128]) * 2
STARTS = [sum(SIZES[:w]) for w in range(NW)]
K_MAX = max(SIZES) // CHUNK

NBUF = 8


def _sc_scatter_body(m_hbm, srcw, dstw, oidxw, mtgtw, zeros_hbm, out_hbm,
                     src_v, dst_v, oidx_v, mtgt_v, fb, acc, stage,
                     *bufs_and_sems):
    bufs = bufs_and_sems[:NBUF]
    sems = bufs_and_sems[NBUF:]
    c = lax.axis_index("c")
    s = lax.axis_index("s")
    wid = c * NS + s
    iota = lax.iota(jnp.int32, 16)
    pltpu.sync_copy(zeros_hbm.at[pl.ds(0, SPAN_MAX)], acc)
    @pl.when(c == 1)
    def _():
        pltpu.sync_copy(zeros_hbm.at[pl.ds(s * SLAB, SLAB)],
                        out_hbm.at[pl.ds(N_PAD + s * SLAB, SLAB)])
    pltpu.sync_copy(srcw.at[wid], src_v)
    pltpu.sync_copy(dstw.at[wid], dst_v)
    pltpu.sync_copy(oidxw.at[wid], oidx_v)
    pltpu.sync_copy(mtgtw.at[wid], mtgt_v)

    def fold(buf, j):
        def group(g, carry):
            dstv = dst_v[pl.ds(j * CHUNK + g * 16, 16)]
            for i in range(16):
                ridx = jnp.full((16,), dstv[i], jnp.int32)
                plsc.addupdate_scatter(acc, [ridx, iota],
                                       buf[g * 16 + i, 0:16])
                plsc.addupdate_scatter(acc, [ridx, iota + 16],
                                       buf[g * 16 + i, 16:32])
            return carry
        lax.fori_loop(0, CHUNK // 16, group, 0)

    for i in range(NBUF):
        pltpu.async_copy(m_hbm.at[src_v.at[i]], bufs[i], sems[i])

    def outer(p, carry):
        for i in range(NBUF):
            j = NBUF * p + i
            pltpu.make_async_copy(m_hbm.at[src_v.at[j]], bufs[i],
                                  sems[i]).wait()
            fold(bufs[i], j)
            @pl.when(j + NBUF < K_MAX)
            def _():
                pltpu.async_copy(m_hbm.at[src_v.at[j + NBUF]], bufs[i],
                                 sems[i])
        return carry

    lax.fori_loop(0, K_MAX // NBUF, outer, 0)
    for i in range(K_MAX % NBUF):
        j = (K_MAX // NBUF) * NBUF + i
        pltpu.make_async_copy(m_hbm.at[src_v.at[j]], bufs[i],
                              sems[i]).wait()
        fold(bufs[i], j)

    pltpu.sync_copy(acc.at[pl.ds(0, 1)], stage.at[pl.ds(s, 1)])
    plsc.subcore_barrier()
    pltpu.sync_copy(stage.at[pl.ds(s + 1, 1)], fb)
    mt = mtgt_v[pl.ds(0, 16)]
    ridx = jnp.full((16,), mt[0], jnp.int32)
    plsc.addupdate_scatter(acc, [ridx, iota], fb[0, 0:16])
    plsc.addupdate_scatter(acc, [ridx, iota + 16], fb[0, 16:32])
    for j in range(SPAN_MAX // CHUNK):
        pltpu.sync_copy(acc.at[pl.ds(j * CHUNK, CHUNK)],
                        out_hbm.at[oidx_v.at[j]])


_SC_SCATTER_CACHE = []


def _sc_scatter(m, srcw, dstw, oidxw, mtgtw, zeros):
    if not _SC_SCATTER_CACHE:
        _SC_SCATTER_CACHE.append(pl.kernel(
            _sc_scatter_body,
            out_type=jax.ShapeDtypeStruct((2 * N_PAD, H), jnp.float32),
            mesh=plsc.VectorSubcoreMesh(core_axis_name="c",
                                        subcore_axis_name="s"),
            scratch_types=[
                pltpu.VMEM((K_MAX, CHUNK), jnp.int32),
                pltpu.VMEM((K_MAX * CHUNK,), jnp.int32),
                pltpu.VMEM((SPAN_MAX // CHUNK, CHUNK), jnp.int32),
                pltpu.VMEM((16,), jnp.int32),
                pltpu.VMEM((1, H), jnp.float32),
                pltpu.VMEM((SPAN_MAX, H), jnp.float32),
                pltpu.VMEM_SHARED((NS + 1, H), jnp.float32),
            ] + [pltpu.VMEM((CHUNK, H), jnp.float32)] * NBUF
              + [pltpu.SemaphoreType.DMA] * NBUF,
            compiler_params=pltpu.CompilerParams(
                use_tc_tiling_on_sc=False, needs_layout_passes=False),
        ))
    return _SC_SCATTER_CACHE[0](m, srcw, dstw, oidxw, mtgtw, zeros)


def _edge_plan(src, dst):
    perm = jnp.argsort(dst, stable=True)
    src_s = src[perm]
    dst_s = dst[perm]
    starts = jnp.asarray(STARTS, jnp.int32)
    sizes = jnp.asarray(SIZES, jnp.int32)
    lo = dst_s[starts]
    hi = dst_s[starts + sizes - 1]
    astart = jnp.concatenate([jnp.zeros((1,), dst_s.dtype), hi[:-1] + 1])
    aend = jnp.concatenate([astart[1:], jnp.asarray([N_PAD], dst_s.dtype)])
    base = jnp.minimum(lo, astart)
    tile = jnp.arange(NW, dtype=jnp.int32) % NS

    base_pe = jnp.repeat(base, sizes, total_repeat_length=E)
    loc = jnp.clip(dst_s - base_pe, 0, SPAN_MAX - 2)

    src_list, dst_list = [], []
    for w in range(NW):
        o, n = STARTS[w], SIZES[w]
        padn = K_MAX * CHUNK - n
        sseg = src_s[o:o + n]
        dseg = loc[o:o + n]
        if padn:
            sseg = jnp.concatenate(
                [sseg, (jnp.arange(padn, dtype=jnp.int32) * 97) % N])
            dseg = jnp.concatenate(
                [dseg, jnp.full((padn,), SPAN_MAX - 1, jnp.int32)])
        src_list.append(sseg)
        dst_list.append(dseg)
    srcw = jnp.stack(src_list).reshape(NW, K_MAX, CHUNK)
    dstw = jnp.stack(dst_list).reshape(NW, K_MAX * CHUNK)

    ar = jnp.arange(SPAN_MAX, dtype=jnp.int32)[None, :]
    r = base[:, None] + ar
    sac = N + (ar % (N_PAD - N))
    oidx = jnp.where((r >= astart[:, None]) & (r < aend[:, None]), r, sac)
    shared_prev = jnp.concatenate(
        [jnp.zeros((1,), jnp.bool_), lo[1:] == hi[:-1]])
    cross = jnp.zeros((NW,), jnp.bool_).at[NS].set(shared_prev[NS])
    oidx = jnp.where(cross[:, None] & (ar == 0), N_PAD + r, oidx)
    oidx = oidx.reshape(NW, SPAN_MAX // CHUNK, CHUNK)

    nxt_same_sc = (jnp.arange(NW) % NS) != (NS - 1)
    flag = nxt_same_sc & jnp.concatenate([lo[1:] == hi[:-1],
                                          jnp.zeros((1,), jnp.bool_)])
    mtgt = jnp.where(flag, jnp.clip(hi - base, 0, SPAN_MAX - 2),
                     SPAN_MAX - 1)
    mtgtw = jnp.broadcast_to(mtgt.astype(jnp.int32)[:, None], (NW, 16))
    return srcw, dstw, oidx.astype(jnp.int32), mtgtw


def _mm_body(x_ref, w_ref, o_ref):
    o_ref[...] = jnp.dot(x_ref[...], w_ref[...],
                         preferred_element_type=jnp.float32)


_mm = pl.pallas_call(
    _mm_body,
    out_shape=jax.ShapeDtypeStruct((N, H), jnp.float32),
)


def _gru_body(h_ref, agg_ref, wr_i, wz_i, wn_i, wr_h, wz_h, wn_h,
              br_i, bz_i, bn_i, br_h, bz_h, bn_h, wnext_ref,
              hout_ref, mout_ref):
    h = h_ref[...]
    agg = agg_ref[:N, :] + agg_ref[N_PAD:N_PAD + N, :]

    def dot(a, b):
        return lax.dot_general(a, b, (((1,), (1,)), ((), ())),
                               preferred_element_type=jnp.float32)

    ir = dot(agg, wr_i[...]) + br_i[...]
    iz = dot(agg, wz_i[...]) + bz_i[...]
    inn = dot(agg, wn_i[...]) + bn_i[...]
    hr = dot(h, wr_h[...]) + br_h[...]
    hz = dot(h, wz_h[...]) + bz_h[...]
    hn = dot(h, wn_h[...]) + bn_h[...]
    r = jax.nn.sigmoid(ir + hr)
    z = jax.nn.sigmoid(iz + hz)
    ng = jnp.tanh(inn + r * hn)
    hnew = (1.0 - z) * ng + z * h
    hout_ref[...] = hnew
    mout_ref[...] = jnp.dot(hnew, wnext_ref[...],
                            preferred_element_type=jnp.float32)


_gru = pl.pallas_call(
    _gru_body,
    out_shape=[jax.ShapeDtypeStruct((N, H), jnp.float32),
               jax.ShapeDtypeStruct((N, H), jnp.float32)],
)


def _pool_body(h_ref, batch_ref, attw_ref, attb_ref, linw_ref, linb_ref,
               o_ref):
    h = h_ref[...]
    b = batch_ref[...]
    seg = lax.broadcasted_iota(jnp.int32, (1, 128), 1)
    m = (b == seg)
    gate = jnp.tanh(jnp.dot(h, attw_ref[...],
                            preferred_element_type=jnp.float32)
                    + attb_ref[...])
    gmax = jnp.max(jnp.where(m, gate, -1e30), axis=0, keepdims=True)
    gmax_sel = jnp.sum(jnp.where(m, gmax, 0.0), axis=1, keepdims=True)
    ge = jnp.exp(gate - gmax_sel)
    denom = jnp.sum(jnp.where(m, ge, 0.0), axis=0, keepdims=True)
    den_sel = jnp.sum(jnp.where(m, denom, 0.0), axis=1, keepdims=True)
    alpha = ge / (den_sel + 1e-16)
    mf = m.astype(jnp.float32)
    pooled = lax.dot_general(mf, alpha * h, (((0,), (0,)), ((), ())),
                             preferred_element_type=jnp.float32)
    out = jnp.dot(pooled, linw_ref[...],
                  preferred_element_type=jnp.float32) + linb_ref[...]
    o_ref[...] = jax.nn.sigmoid(out)


_pool = pl.pallas_call(
    _pool_body,
    out_shape=jax.ShapeDtypeStruct((128, 1), jnp.float32),
)


def kernel(x, edge_index, batch, W1, gru1_wih, gru1_whh, gru1_bih, gru1_bhh,
           W2, gru2_wih, gru2_whh, gru2_bih, gru2_bhh,
           att_gate_w, att_gate_b, lin_w, lin_b):
    f32 = jnp.float32
    src = edge_index[0]
    dst = edge_index[1]
    srcw, dstw, oidxw, mtgtw = _edge_plan(src, dst)
    zeros = jnp.zeros((N_PAD, H), dtype=f32)

    def gru_weights(wih, whh, bih, bhh):
        ws = tuple(wih[i * H:(i + 1) * H] for i in range(3)) + \
             tuple(whh[i * H:(i + 1) * H] for i in range(3))
        bs = tuple(bih[i * H:(i + 1) * H].reshape(1, H) for i in range(3)) + \
             tuple(bhh[i * H:(i + 1) * H].reshape(1, H) for i in range(3))
        return ws + bs

    g1 = gru_weights(gru1_wih, gru1_whh, gru1_bih, gru1_bhh)
    g2 = gru_weights(gru2_wih, gru2_whh, gru2_bih, gru2_bhh)
    wnext1 = jnp.concatenate([W1[1:], W2[:1]], axis=0)
    wnext2 = jnp.concatenate([W2[1:], W2[:1]], axis=0)

    def layer(carry_in, gw, wnext):
        def body(t, carry):
            h, m = carry
            aggs = _sc_scatter(m, srcw, dstw, oidxw, mtgtw, zeros)
            wn = lax.dynamic_index_in_dim(wnext, t, 0, keepdims=False)
            h, m = _gru(h, aggs, *gw, wn)
            return (h, m)
        return lax.fori_loop(0, L, body, carry_in)

    m0 = _mm(x, W1[0])
    h, m = layer((x, m0), g1, wnext1)
    h, m = layer((h, m), g2, wnext2)

    out = _pool(h, batch.reshape(N, 1),
                att_gate_w.reshape(H, 1), att_gate_b.reshape(1, 1),
                lin_w.reshape(H, 1), lin_b.reshape(1, 1))
    return out[:G]


# speedup vs baseline: 1.0385x; 1.0385x over previous
import functools

import jax
import jax.numpy as jnp
from jax import lax
from jax.experimental import pallas as pl
from jax.experimental.pallas import tpu as pltpu
from jax.experimental.pallas import tpu_sc as plsc

N = 10000
E = 640000
H = 32
L = 32
G = 64

NC = 2
NS = 16
NW = NC * NS
CHUNK = 128
N_PAD = 10112
SLAB = N_PAD // NS
SPAN_MAX = 1024
SIZES = ([159 * 128, 159 * 128] + [156 * 128] * 13 + [154 * 128]) * 2
STARTS = [sum(SIZES[:w]) for w in range(NW)]
K_MAX = max(SIZES) // CHUNK

NBUF = 4


def _sc_scatter_body(m_hbm, srcw, dstw, samew, oidxw, mtgtw, zeros_hbm,
                     out_hbm,
                     src_v, dst_v, same_v, oidx_v, mtgt_v, fb, acc, stage,
                     *bufs_and_sems):
    bufs = bufs_and_sems[:NBUF]
    sems = bufs_and_sems[NBUF:]
    c = lax.axis_index("c")
    s = lax.axis_index("s")
    wid = c * NS + s
    iota = lax.iota(jnp.int32, 16)
    pltpu.sync_copy(zeros_hbm.at[pl.ds(0, SPAN_MAX)], acc)
    @pl.when(c == 1)
    def _():
        pltpu.sync_copy(zeros_hbm.at[pl.ds(s * SLAB, SLAB)],
                        out_hbm.at[pl.ds(N_PAD + s * SLAB, SLAB)])
    pltpu.sync_copy(srcw.at[wid], src_v)
    pltpu.sync_copy(dstw.at[wid], dst_v)
    pltpu.sync_copy(samew.at[wid], same_v)
    pltpu.sync_copy(oidxw.at[wid], oidx_v)
    pltpu.sync_copy(mtgtw.at[wid], mtgt_v)

    def fold(buf, j, carry):
        def group(g, carry):
            a0, a1 = carry
            base = j * CHUNK + g * 16
            dstv = dst_v[pl.ds(base, 16)]
            samev = same_v[pl.ds(base, 16)]
            for i in range(16):
                sb = jnp.full((16,), samev[i], jnp.float32)
                rb = jnp.full((16,), dstv[i], jnp.int32)
                a0 = a0 * sb + buf[g * 16 + i, 0:16]
                a1 = a1 * sb + buf[g * 16 + i, 16:32]
                plsc.store_scatter(acc, [rb, iota], a0)
                plsc.store_scatter(acc, [rb, iota + 16], a1)
            return (a0, a1)
        return lax.fori_loop(0, CHUNK // 16, group, carry)

    for i in range(NBUF):
        pltpu.async_copy(m_hbm.at[src_v.at[i]], bufs[i], sems[i])

    zf = jnp.zeros((16,), jnp.float32)

    def outer(p, carry):
        for i in range(NBUF):
            j = NBUF * p + i
            pltpu.make_async_copy(m_hbm.at[src_v.at[j]], bufs[i],
                                  sems[i]).wait()
            carry = fold(bufs[i], j, carry)
            @pl.when(j + NBUF < K_MAX)
            def _():
                pltpu.async_copy(m_hbm.at[src_v.at[j + NBUF]], bufs[i],
                                 sems[i])
        return carry

    carry = lax.fori_loop(0, K_MAX // NBUF, outer, (zf, zf))
    for i in range(K_MAX % NBUF):
        j = (K_MAX // NBUF) * NBUF + i
        pltpu.make_async_copy(m_hbm.at[src_v.at[j]], bufs[i],
                              sems[i]).wait()
        carry = fold(bufs[i], j, carry)

    pltpu.sync_copy(acc.at[pl.ds(0, 1)], stage.at[pl.ds(s, 1)])
    plsc.subcore_barrier()
    pltpu.sync_copy(stage.at[pl.ds(s + 1, 1)], fb)
    mt = mtgt_v[pl.ds(0, 16)]
    ridx = jnp.full((16,), mt[0], jnp.int32)
    plsc.addupdate_scatter(acc, [ridx, iota], fb[0, 0:16])
    plsc.addupdate_scatter(acc, [ridx, iota + 16], fb[0, 16:32])
    for j in range(SPAN_MAX // CHUNK):
        pltpu.sync_copy(acc.at[pl.ds(j * CHUNK, CHUNK)],
                        out_hbm.at[oidx_v.at[j]])


_SC_SCATTER_CACHE = []


def _sc_scatter(m, srcw, dstw, samew, oidxw, mtgtw, zeros):
    if not _SC_SCATTER_CACHE:
        _SC_SCATTER_CACHE.append(pl.kernel(
            _sc_scatter_body,
            out_type=jax.ShapeDtypeStruct((2 * N_PAD, H), jnp.float32),
            mesh=plsc.VectorSubcoreMesh(core_axis_name="c",
                                        subcore_axis_name="s"),
            scratch_types=[
                pltpu.VMEM((K_MAX, CHUNK), jnp.int32),
                pltpu.VMEM((K_MAX * CHUNK,), jnp.int32),
                pltpu.VMEM((K_MAX * CHUNK,), jnp.float32),
                pltpu.VMEM((SPAN_MAX // CHUNK, CHUNK), jnp.int32),
                pltpu.VMEM((16,), jnp.int32),
                pltpu.VMEM((1, H), jnp.float32),
                pltpu.VMEM((SPAN_MAX, H), jnp.float32),
                pltpu.VMEM_SHARED((NS + 1, H), jnp.float32),
            ] + [pltpu.VMEM((CHUNK, H), jnp.float32)] * NBUF
              + [pltpu.SemaphoreType.DMA] * NBUF,
            compiler_params=pltpu.CompilerParams(
                use_tc_tiling_on_sc=False, needs_layout_passes=False),
        ))
    return _SC_SCATTER_CACHE[0](m, srcw, dstw, samew, oidxw, mtgtw, zeros)


def _edge_plan(src, dst):
    perm = jnp.argsort(dst, stable=True)
    src_s = src[perm]
    dst_s = dst[perm]
    starts = jnp.asarray(STARTS, jnp.int32)
    sizes = jnp.asarray(SIZES, jnp.int32)
    lo = dst_s[starts]
    hi = dst_s[starts + sizes - 1]
    astart = jnp.concatenate([jnp.zeros((1,), dst_s.dtype), hi[:-1] + 1])
    aend = jnp.concatenate([astart[1:], jnp.asarray([N_PAD], dst_s.dtype)])
    base = jnp.minimum(lo, astart)
    tile = jnp.arange(NW, dtype=jnp.int32) % NS

    base_pe = jnp.repeat(base, sizes, total_repeat_length=E)
    loc = jnp.clip(dst_s - base_pe, 0, SPAN_MAX - 2)

    same = jnp.concatenate([jnp.zeros((1,), jnp.bool_),
                            dst_s[1:] == dst_s[:-1]])
    wstart = jnp.zeros((E,), jnp.bool_).at[starts].set(True)
    same = (same & ~wstart).astype(jnp.float32)

    src_list, dst_list, same_list = [], [], []
    for w in range(NW):
        o, n = STARTS[w], SIZES[w]
        padn = K_MAX * CHUNK - n
        sseg = src_s[o:o + n]
        dseg = loc[o:o + n]
        mseg = same[o:o + n]
        if padn:
            sseg = jnp.concatenate(
                [sseg, (jnp.arange(padn, dtype=jnp.int32) * 97) % N])
            dseg = jnp.concatenate(
                [dseg, jnp.full((padn,), SPAN_MAX - 1, jnp.int32)])
            mseg = jnp.concatenate([mseg, jnp.zeros((padn,), jnp.float32)])
        src_list.append(sseg)
        dst_list.append(dseg)
        same_list.append(mseg)
    srcw = jnp.stack(src_list).reshape(NW, K_MAX, CHUNK)
    dstw = jnp.stack(dst_list).reshape(NW, K_MAX * CHUNK)
    samew = jnp.stack(same_list).reshape(NW, K_MAX * CHUNK)

    ar = jnp.arange(SPAN_MAX, dtype=jnp.int32)[None, :]
    r = base[:, None] + ar
    sac = N + (ar % (N_PAD - N))
    oidx = jnp.where((r >= astart[:, None]) & (r < aend[:, None]), r, sac)
    shared_prev = jnp.concatenate(
        [jnp.zeros((1,), jnp.bool_), lo[1:] == hi[:-1]])
    cross = jnp.zeros((NW,), jnp.bool_).at[NS].set(shared_prev[NS])
    oidx = jnp.where(cross[:, None] & (ar == 0), N_PAD + r, oidx)
    oidx = oidx.reshape(NW, SPAN_MAX // CHUNK, CHUNK)

    nxt_same_sc = (jnp.arange(NW) % NS) != (NS - 1)
    flag = nxt_same_sc & jnp.concatenate([lo[1:] == hi[:-1],
                                          jnp.zeros((1,), jnp.bool_)])
    mtgt = jnp.where(flag, jnp.clip(hi - base, 0, SPAN_MAX - 2),
                     SPAN_MAX - 1)
    mtgtw = jnp.broadcast_to(mtgt.astype(jnp.int32)[:, None], (NW, 16))
    return srcw, dstw, samew, oidx.astype(jnp.int32), mtgtw


def _mm_body(x_ref, w_ref, o_ref):
    o_ref[...] = jnp.dot(x_ref[...], w_ref[...],
                         preferred_element_type=jnp.float32)


_mm = pl.pallas_call(
    _mm_body,
    out_shape=jax.ShapeDtypeStruct((N, H), jnp.float32),
)


def _gru_body(h_ref, agg_ref, wr_i, wz_i, wn_i, wr_h, wz_h, wn_h,
              br_i, bz_i, bn_i, br_h, bz_h, bn_h, wnext_ref,
              hout_ref, mout_ref):
    h = h_ref[...]
    agg = agg_ref[:N, :] + agg_ref[N_PAD:N_PAD + N, :]

    def dot(a, b):
        return lax.dot_general(a, b, (((1,), (1,)), ((), ())),
                               preferred_element_type=jnp.float32)

    ir = dot(agg, wr_i[...]) + br_i[...]
    iz = dot(agg, wz_i[...]) + bz_i[...]
    inn = dot(agg, wn_i[...]) + bn_i[...]
    hr = dot(h, wr_h[...]) + br_h[...]
    hz = dot(h, wz_h[...]) + bz_h[...]
    hn = dot(h, wn_h[...]) + bn_h[...]
    r = jax.nn.sigmoid(ir + hr)
    z = jax.nn.sigmoid(iz + hz)
    ng = jnp.tanh(inn + r * hn)
    hnew = (1.0 - z) * ng + z * h
    hout_ref[...] = hnew
    mout_ref[...] = jnp.dot(hnew, wnext_ref[...],
                            preferred_element_type=jnp.float32)


_gru = pl.pallas_call(
    _gru_body,
    out_shape=[jax.ShapeDtypeStruct((N, H), jnp.float32),
               jax.ShapeDtypeStruct((N, H), jnp.float32)],
)


def _pool_body(h_ref, batch_ref, attw_ref, attb_ref, linw_ref, linb_ref,
               o_ref):
    h = h_ref[...]
    b = batch_ref[...]
    seg = lax.broadcasted_iota(jnp.int32, (1, 128), 1)
    m = (b == seg)
    gate = jnp.tanh(jnp.dot(h, attw_ref[...],
                            preferred_element_type=jnp.float32)
                    + attb_ref[...])
    gmax = jnp.max(jnp.where(m, gate, -1e30), axis=0, keepdims=True)
    gmax_sel = jnp.sum(jnp.where(m, gmax, 0.0), axis=1, keepdims=True)
    ge = jnp.exp(gate - gmax_sel)
    denom = jnp.sum(jnp.where(m, ge, 0.0), axis=0, keepdims=True)
    den_sel = jnp.sum(jnp.where(m, denom, 0.0), axis=1, keepdims=True)
    alpha = ge / (den_sel + 1e-16)
    mf = m.astype(jnp.float32)
    pooled = lax.dot_general(mf, alpha * h, (((0,), (0,)), ((), ())),
                             preferred_element_type=jnp.float32)
    out = jnp.dot(pooled, linw_ref[...],
                  preferred_element_type=jnp.float32) + linb_ref[...]
    o_ref[...] = jax.nn.sigmoid(out)


_pool = pl.pallas_call(
    _pool_body,
    out_shape=jax.ShapeDtypeStruct((128, 1), jnp.float32),
)


def kernel(x, edge_index, batch, W1, gru1_wih, gru1_whh, gru1_bih, gru1_bhh,
           W2, gru2_wih, gru2_whh, gru2_bih, gru2_bhh,
           att_gate_w, att_gate_b, lin_w, lin_b):
    f32 = jnp.float32
    src = edge_index[0]
    dst = edge_index[1]
    srcw, dstw, samew, oidxw, mtgtw = _edge_plan(src, dst)
    zeros = jnp.zeros((N_PAD, H), dtype=f32)

    def gru_weights(wih, whh, bih, bhh):
        ws = tuple(wih[i * H:(i + 1) * H] for i in range(3)) + \
             tuple(whh[i * H:(i + 1) * H] for i in range(3))
        bs = tuple(bih[i * H:(i + 1) * H].reshape(1, H) for i in range(3)) + \
             tuple(bhh[i * H:(i + 1) * H].reshape(1, H) for i in range(3))
        return ws + bs

    g1 = gru_weights(gru1_wih, gru1_whh, gru1_bih, gru1_bhh)
    g2 = gru_weights(gru2_wih, gru2_whh, gru2_bih, gru2_bhh)
    wnext1 = jnp.concatenate([W1[1:], W2[:1]], axis=0)
    wnext2 = jnp.concatenate([W2[1:], W2[:1]], axis=0)

    def layer(carry_in, gw, wnext):
        def body(t, carry):
            h, m = carry
            aggs = _sc_scatter(m, srcw, dstw, samew, oidxw, mtgtw, zeros)
            wn = lax.dynamic_index_in_dim(wnext, t, 0, keepdims=False)
            h, m = _gru(h, aggs, *gw, wn)
            return (h, m)
        return lax.fori_loop(0, L, body, carry_in)

    m0 = _mm(x, W1[0])
    h, m = layer((x, m0), g1, wnext1)
    h, m = layer((h, m), g2, wnext2)

    out = _pool(h, batch.reshape(N, 1),
                att_gate_w.reshape(H, 1), att_gate_b.reshape(1, 1),
                lin_w.reshape(H, 1), lin_b.reshape(1, 1))
    return out[:G]


# speedup vs baseline: 1.1959x; 1.1515x over previous
import functools

import jax
import jax.numpy as jnp
from jax import lax
from jax.experimental import pallas as pl
from jax.experimental.pallas import tpu as pltpu
from jax.experimental.pallas import tpu_sc as plsc

N = 10000
E = 640000
H = 32
L = 32
G = 64

NC = 2
NS = 16
NW = NC * NS
CHUNK = 128
N_PAD = 10112
SLAB = N_PAD // NS
SPAN_MAX = 1024
SIZES = ([159 * 128, 159 * 128] + [156 * 128] * 13 + [154 * 128]) * 2
STARTS = [sum(SIZES[:w]) for w in range(NW)]
K_MAX = max(SIZES) // CHUNK

NBUF = 2


def _sc_scatter_body(m_hbm, srcw, dstw, samew, oidxw, mtgtw, zeros_hbm,
                     out_hbm,
                     src_v, dst_v, same_v, oidx_v, mtgt_v, fb, acc, stage,
                     m_sh, *bufs_and_sems):
    bufs = bufs_and_sems[:NBUF]
    sems = bufs_and_sems[NBUF:]
    c = lax.axis_index("c")
    s = lax.axis_index("s")
    wid = c * NS + s
    iota = lax.iota(jnp.int32, 16)
    pltpu.sync_copy(zeros_hbm.at[pl.ds(0, SPAN_MAX)], acc)
    @pl.when(c == 1)
    def _():
        pltpu.sync_copy(zeros_hbm.at[pl.ds(s * SLAB, SLAB)],
                        out_hbm.at[pl.ds(N_PAD + s * SLAB, SLAB)])
    pltpu.sync_copy(srcw.at[wid], src_v)
    pltpu.sync_copy(dstw.at[wid], dst_v)
    pltpu.sync_copy(samew.at[wid], same_v)
    pltpu.sync_copy(oidxw.at[wid], oidx_v)
    pltpu.sync_copy(mtgtw.at[wid], mtgt_v)
    pltpu.sync_copy(m_hbm.at[pl.ds(s * (N // NS), N // NS)],
                    m_sh.at[pl.ds(s * (N // NS), N // NS)])
    plsc.subcore_barrier()

    def fold(buf, j, carry):
        def group(g, carry):
            a0, a1 = carry
            base = j * CHUNK + g * 16
            dstv = dst_v[pl.ds(base, 16)]
            samev = same_v[pl.ds(base, 16)]
            for i in range(16):
                sb = jnp.full((16,), samev[i], jnp.float32)
                rb = jnp.full((16,), dstv[i], jnp.int32)
                a0 = a0 * sb + buf[g * 16 + i, 0:16]
                a1 = a1 * sb + buf[g * 16 + i, 16:32]
                plsc.store_scatter(acc, [rb, iota], a0)
                plsc.store_scatter(acc, [rb, iota + 16], a1)
            return (a0, a1)
        return lax.fori_loop(0, CHUNK // 16, group, carry)

    for i in range(NBUF):
        pltpu.async_copy(m_sh.at[src_v.at[i]], bufs[i], sems[i])

    zf = jnp.zeros((16,), jnp.float32)

    def outer(p, carry):
        for i in range(NBUF):
            j = NBUF * p + i
            pltpu.make_async_copy(m_sh.at[src_v.at[j]], bufs[i],
                                  sems[i]).wait()
            carry = fold(bufs[i], j, carry)
            @pl.when(j + NBUF < K_MAX)
            def _():
                pltpu.async_copy(m_sh.at[src_v.at[j + NBUF]], bufs[i],
                                 sems[i])
        return carry

    carry = lax.fori_loop(0, K_MAX // NBUF, outer, (zf, zf))
    for i in range(K_MAX % NBUF):
        j = (K_MAX // NBUF) * NBUF + i
        pltpu.make_async_copy(m_sh.at[src_v.at[j]], bufs[i],
                              sems[i]).wait()
        carry = fold(bufs[i], j, carry)

    pltpu.sync_copy(acc.at[pl.ds(0, 1)], stage.at[pl.ds(s, 1)])
    plsc.subcore_barrier()
    pltpu.sync_copy(stage.at[pl.ds(s + 1, 1)], fb)
    mt = mtgt_v[pl.ds(0, 16)]
    ridx = jnp.full((16,), mt[0], jnp.int32)
    plsc.addupdate_scatter(acc, [ridx, iota], fb[0, 0:16])
    plsc.addupdate_scatter(acc, [ridx, iota + 16], fb[0, 16:32])
    for j in range(SPAN_MAX // CHUNK):
        pltpu.sync_copy(acc.at[pl.ds(j * CHUNK, CHUNK)],
                        out_hbm.at[oidx_v.at[j]])


_SC_SCATTER_CACHE = []


def _sc_scatter(m, srcw, dstw, samew, oidxw, mtgtw, zeros):
    if not _SC_SCATTER_CACHE:
        _SC_SCATTER_CACHE.append(pl.kernel(
            _sc_scatter_body,
            out_type=jax.ShapeDtypeStruct((2 * N_PAD, H), jnp.float32),
            mesh=plsc.VectorSubcoreMesh(core_axis_name="c",
                                        subcore_axis_name="s"),
            scratch_types=[
                pltpu.VMEM((K_MAX, CHUNK), jnp.int32),
                pltpu.VMEM((K_MAX * CHUNK,), jnp.int32),
                pltpu.VMEM((K_MAX * CHUNK,), jnp.float32),
                pltpu.VMEM((SPAN_MAX // CHUNK, CHUNK), jnp.int32),
                pltpu.VMEM((16,), jnp.int32),
                pltpu.VMEM((1, H), jnp.float32),
                pltpu.VMEM((SPAN_MAX, H), jnp.float32),
                pltpu.VMEM_SHARED((NS + 1, H), jnp.float32),
                pltpu.VMEM_SHARED((N, H), jnp.float32),
            ] + [pltpu.VMEM((CHUNK, H), jnp.float32)] * NBUF
              + [pltpu.SemaphoreType.DMA] * NBUF,
            compiler_params=pltpu.CompilerParams(
                use_tc_tiling_on_sc=False, needs_layout_passes=False),
        ))
    return _SC_SCATTER_CACHE[0](m, srcw, dstw, samew, oidxw, mtgtw, zeros)


def _edge_plan(src, dst):
    perm = jnp.argsort(dst, stable=True)
    src_s = src[perm]
    dst_s = dst[perm]
    starts = jnp.asarray(STARTS, jnp.int32)
    sizes = jnp.asarray(SIZES, jnp.int32)
    lo = dst_s[starts]
    hi = dst_s[starts + sizes - 1]
    astart = jnp.concatenate([jnp.zeros((1,), dst_s.dtype), hi[:-1] + 1])
    aend = jnp.concatenate([astart[1:], jnp.asarray([N_PAD], dst_s.dtype)])
    base = jnp.minimum(lo, astart)
    tile = jnp.arange(NW, dtype=jnp.int32) % NS

    base_pe = jnp.repeat(base, sizes, total_repeat_length=E)
    loc = jnp.clip(dst_s - base_pe, 0, SPAN_MAX - 2)

    same = jnp.concatenate([jnp.zeros((1,), jnp.bool_),
                            dst_s[1:] == dst_s[:-1]])
    wstart = jnp.zeros((E,), jnp.bool_).at[starts].set(True)
    same = (same & ~wstart).astype(jnp.float32)

    src_list, dst_list, same_list = [], [], []
    for w in range(NW):
        o, n = STARTS[w], SIZES[w]
        padn = K_MAX * CHUNK - n
        sseg = src_s[o:o + n]
        dseg = loc[o:o + n]
        mseg = same[o:o + n]
        if padn:
            sseg = jnp.concatenate(
                [sseg, (jnp.arange(padn, dtype=jnp.int32) * 97) % N])
            dseg = jnp.concatenate(
                [dseg, jnp.full((padn,), SPAN_MAX - 1, jnp.int32)])
            mseg = jnp.concatenate([mseg, jnp.zeros((padn,), jnp.float32)])
        src_list.append(sseg)
        dst_list.append(dseg)
        same_list.append(mseg)
    srcw = jnp.stack(src_list).reshape(NW, K_MAX, CHUNK)
    dstw = jnp.stack(dst_list).reshape(NW, K_MAX * CHUNK)
    samew = jnp.stack(same_list).reshape(NW, K_MAX * CHUNK)

    ar = jnp.arange(SPAN_MAX, dtype=jnp.int32)[None, :]
    r = base[:, None] + ar
    sac = N + (ar % (N_PAD - N))
    oidx = jnp.where((r >= astart[:, None]) & (r < aend[:, None]), r, sac)
    shared_prev = jnp.concatenate(
        [jnp.zeros((1,), jnp.bool_), lo[1:] == hi[:-1]])
    cross = jnp.zeros((NW,), jnp.bool_).at[NS].set(shared_prev[NS])
    oidx = jnp.where(cross[:, None] & (ar == 0), N_PAD + r, oidx)
    oidx = oidx.reshape(NW, SPAN_MAX // CHUNK, CHUNK)

    nxt_same_sc = (jnp.arange(NW) % NS) != (NS - 1)
    flag = nxt_same_sc & jnp.concatenate([lo[1:] == hi[:-1],
                                          jnp.zeros((1,), jnp.bool_)])
    mtgt = jnp.where(flag, jnp.clip(hi - base, 0, SPAN_MAX - 2),
                     SPAN_MAX - 1)
    mtgtw = jnp.broadcast_to(mtgt.astype(jnp.int32)[:, None], (NW, 16))
    return srcw, dstw, samew, oidx.astype(jnp.int32), mtgtw


def _mm_body(x_ref, w_ref, o_ref):
    o_ref[...] = jnp.dot(x_ref[...], w_ref[...],
                         preferred_element_type=jnp.float32)


_mm = pl.pallas_call(
    _mm_body,
    out_shape=jax.ShapeDtypeStruct((N, H), jnp.float32),
)


def _gru_body(h_ref, agg_ref, wr_i, wz_i, wn_i, wr_h, wz_h, wn_h,
              br_i, bz_i, bn_i, br_h, bz_h, bn_h, wnext_ref,
              hout_ref, mout_ref):
    h = h_ref[...]
    agg = agg_ref[:N, :] + agg_ref[N_PAD:N_PAD + N, :]

    def dot(a, b):
        return lax.dot_general(a, b, (((1,), (1,)), ((), ())),
                               preferred_element_type=jnp.float32)

    ir = dot(agg, wr_i[...]) + br_i[...]
    iz = dot(agg, wz_i[...]) + bz_i[...]
    inn = dot(agg, wn_i[...]) + bn_i[...]
    hr = dot(h, wr_h[...]) + br_h[...]
    hz = dot(h, wz_h[...]) + bz_h[...]
    hn = dot(h, wn_h[...]) + bn_h[...]
    r = jax.nn.sigmoid(ir + hr)
    z = jax.nn.sigmoid(iz + hz)
    ng = jnp.tanh(inn + r * hn)
    hnew = (1.0 - z) * ng + z * h
    hout_ref[...] = hnew
    mout_ref[...] = jnp.dot(hnew, wnext_ref[...],
                            preferred_element_type=jnp.float32)


_gru = pl.pallas_call(
    _gru_body,
    out_shape=[jax.ShapeDtypeStruct((N, H), jnp.float32),
               jax.ShapeDtypeStruct((N, H), jnp.float32)],
)


def _pool_body(h_ref, batch_ref, attw_ref, attb_ref, linw_ref, linb_ref,
               o_ref):
    h = h_ref[...]
    b = batch_ref[...]
    seg = lax.broadcasted_iota(jnp.int32, (1, 128), 1)
    m = (b == seg)
    gate = jnp.tanh(jnp.dot(h, attw_ref[...],
                            preferred_element_type=jnp.float32)
                    + attb_ref[...])
    gmax = jnp.max(jnp.where(m, gate, -1e30), axis=0, keepdims=True)
    gmax_sel = jnp.sum(jnp.where(m, gmax, 0.0), axis=1, keepdims=True)
    ge = jnp.exp(gate - gmax_sel)
    denom = jnp.sum(jnp.where(m, ge, 0.0), axis=0, keepdims=True)
    den_sel = jnp.sum(jnp.where(m, denom, 0.0), axis=1, keepdims=True)
    alpha = ge / (den_sel + 1e-16)
    mf = m.astype(jnp.float32)
    pooled = lax.dot_general(mf, alpha * h, (((0,), (0,)), ((), ())),
                             preferred_element_type=jnp.float32)
    out = jnp.dot(pooled, linw_ref[...],
                  preferred_element_type=jnp.float32) + linb_ref[...]
    o_ref[...] = jax.nn.sigmoid(out)


_pool = pl.pallas_call(
    _pool_body,
    out_shape=jax.ShapeDtypeStruct((128, 1), jnp.float32),
)


def kernel(x, edge_index, batch, W1, gru1_wih, gru1_whh, gru1_bih, gru1_bhh,
           W2, gru2_wih, gru2_whh, gru2_bih, gru2_bhh,
           att_gate_w, att_gate_b, lin_w, lin_b):
    f32 = jnp.float32
    src = edge_index[0]
    dst = edge_index[1]
    srcw, dstw, samew, oidxw, mtgtw = _edge_plan(src, dst)
    zeros = jnp.zeros((N_PAD, H), dtype=f32)

    def gru_weights(wih, whh, bih, bhh):
        ws = tuple(wih[i * H:(i + 1) * H] for i in range(3)) + \
             tuple(whh[i * H:(i + 1) * H] for i in range(3))
        bs = tuple(bih[i * H:(i + 1) * H].reshape(1, H) for i in range(3)) + \
             tuple(bhh[i * H:(i + 1) * H].reshape(1, H) for i in range(3))
        return ws + bs

    g1 = gru_weights(gru1_wih, gru1_whh, gru1_bih, gru1_bhh)
    g2 = gru_weights(gru2_wih, gru2_whh, gru2_bih, gru2_bhh)
    wnext1 = jnp.concatenate([W1[1:], W2[:1]], axis=0)
    wnext2 = jnp.concatenate([W2[1:], W2[:1]], axis=0)

    def layer(carry_in, gw, wnext):
        def body(t, carry):
            h, m = carry
            aggs = _sc_scatter(m, srcw, dstw, samew, oidxw, mtgtw, zeros)
            wn = lax.dynamic_index_in_dim(wnext, t, 0, keepdims=False)
            h, m = _gru(h, aggs, *gw, wn)
            return (h, m)
        return lax.fori_loop(0, L, body, carry_in)

    m0 = _mm(x, W1[0])
    h, m = layer((x, m0), g1, wnext1)
    h, m = layer((h, m), g2, wnext2)

    out = _pool(h, batch.reshape(N, 1),
                att_gate_w.reshape(H, 1), att_gate_b.reshape(1, 1),
                lin_w.reshape(H, 1), lin_b.reshape(1, 1))
    return out[:G]
